# baseline (speedup 1.0000x reference)
import functools
import jax
import jax.numpy as jnp
from jax import lax
from jax.experimental import pallas as pl
from jax.experimental.pallas import tpu as pltpu
from jax.experimental.pallas import tpu_sc as plsc

_H = 228
_L = 56
_CCH = 4
_NST = 32 // _CCH
_NBAND = 29


def _mm(a, b):
    return jax.lax.dot_general(
        a, b, (((1,), (0,)), ((), ())),
        precision=jax.lax.Precision.HIGHEST,
        preferred_element_type=jnp.float32)


def _selmat():
    r = jax.lax.broadcasted_iota(jnp.int32, (_L, 226), 1)
    lh = jax.lax.broadcasted_iota(jnp.int32, (_L, 226), 0)
    d = r - 4 * lh
    return ((d == 0) | (d == 2) | (d == 4)).astype(jnp.float32)


def _foldmat():
    h = jax.lax.broadcasted_iota(jnp.int32, (_H, _L), 0)
    lh = jax.lax.broadcasted_iota(jnp.int32, (_H, _L), 1)
    d = h - 4 * lh
    return ((d >= 0) & (d <= 4)).astype(jnp.float32)


def _boxsum(R, A):
    return _mm(_mm(A, R), A.T)


def _expand(M, P):
    return _mm(_mm(P, M), P.T)


def _sim_body(content_ref, mask_ref, cond_ref, simi_ref, code_ref, acc_ref):
    i = pl.program_id(0)

    @pl.when(i == 0)
    def _():
        acc_ref[...] = jnp.zeros_like(acc_ref)

    c0 = content_ref[...]
    m0 = mask_ref[...]
    cd = cond_ref[...]
    mp = (m0 > 0).astype(jnp.float32)
    cm = c0 * mp
    c2 = cd * cd

    acc_ref[18] += jnp.sum(c0 * c0, axis=0)[:226, :226]
    for n in range(9):
        sh, sw = n // 3, n % 3
        cs = cd[:, sh:sh + 226, sw:sw + 226]
        acc_ref[n] += jnp.sum(cm[:, :226, :226] * cs, axis=0)
        acc_ref[9 + n] += jnp.sum(
            mp[:, :226, :226] * c2[:, sh:sh + 226, sw:sw + 226], axis=0)

    @pl.when(i == _NST - 1)
    def _():
        A = _selmat()
        P = _foldmat()
        eps = jnp.float32(1e-8)
        V = _boxsum(acc_ref[18], A)
        sqV = jnp.sqrt(V) + eps
        nvv = V / (sqV * sqV)
        best_val = jnp.full((_L, _L), -jnp.inf, jnp.float32)
        best_idx = jnp.zeros((_L, _L), jnp.int32)
        for n in range(9):
            D = _boxsum(acc_ref[n], A)
            U = _boxsum(acc_ref[9 + n], A)
            sqU = jnp.sqrt(U) + eps
            nuu = U / (sqU * sqU)
            nvu = D / (sqV * sqU)
            eud = jnp.sqrt(jnp.maximum(nvv + nuu - 2.0 * nvu, 0.0))
            sim = (2.0 - eud) * 0.5
            upd = sim > best_val
            best_val = jnp.where(upd, sim, best_val)
            best_idx = jnp.where(upd, n, best_idx)

        maxval = best_val
        maxidx = jnp.where(maxval > 0, best_idx, 0)

        idx_f = maxidx.astype(jnp.float32)
        pieces = []
        for i2 in range(8):
            idx_s = idx_f[:, i2 * 7:(i2 + 1) * 7]
            sa = idx_f[:, i2 * 4:min((i2 + 1) * 7, _L)]
            ssum = jnp.sum(sa)
            scnt = jnp.sum(sa > 0.0).astype(jnp.float32)
            smean = jnp.round(ssum / (scnt + jnp.float32(1e-8)))
            pieces.append(jnp.where(idx_s > 0.0, idx_s, smean))
        fidx = jnp.concatenate(pieces, axis=1).astype(jnp.int32)

        code_ref[...] = (fidx // 3) * 4 + (fidx % 3)

        cnt = jnp.maximum(_expand(jnp.ones((_L, _L), jnp.float32), P), 1.0)
        simi_ref[...] = _expand(maxval, P) / cnt


def _sc_fold_body(cond_hbm, code_hbm, out_hbm,
                  code_v, cond_v, out_v, idxr_v, idxc_v, msk_v, invc_v, sem):
    wid = lax.axis_index("s") * 2 + lax.axis_index("c")

    @pl.when(wid < _NBAND)
    def _():
        b0 = wid * 8
        r0 = jnp.minimum(b0, 218)
        l0 = jnp.clip(2 * wid - 1, 0, 53)

        pltpu.async_copy(code_hbm.at[pl.ds(l0 * _L, 3 * _L)], code_v, sem).wait()

        def build(hk, carry):
            hh = hk // 15
            k = hk % 15
            h = b0 + hh
            sl = pl.ds(k * 16, 16)
            wv = lax.iota(jnp.int32, 16) + k * 16
            lh0 = h >> 2
            cnt = jnp.zeros((16,), jnp.float32)
            for ab in range(4):
                a, b = ab >> 1, ab & 1
                la = lh0 - a
                vr = (la >= 0) & (la <= 55) & ((h & 3) + 4 * a <= 4)
                lw = (wv >> 2) - b
                vc = (lw >= 0) & (lw <= 55) & ((wv & 3) + 4 * b <= 4)
                valid = vc & vr
                pos = jnp.clip((la - l0) * _L + lw, 0, 3 * _L - 1)
                code = plsc.load_gather(code_v, [pos])
                rvec = jnp.clip((h - r0) + (code >> 2), 0, 9)
                cvec = jnp.clip(wv + (code & 3), 0, 227)
                mf = jnp.where(valid, jnp.float32(1.0), jnp.float32(0.0))
                idxr_v[ab, hh, sl] = rvec
                idxc_v[ab, hh, sl] = cvec
                msk_v[ab, hh, sl] = mf
                cnt = cnt + mf
            invc_v[hh, sl] = 1.0 / jnp.maximum(cnt, 1.0)
            return carry
        lax.fori_loop(0, 120, build, 0)

        for half in range(2):
            ch0 = half * 16
            cps = [pltpu.async_copy(
                cond_hbm.at[ch0 + ci, pl.ds(r0, 10), :], cond_v.at[ci], sem)
                for ci in range(16)]
            for cp in cps:
                cp.wait()

            def gath(hk, carry):
                hh = hk // 15
                k = hk % 15
                sl = pl.ds(k * 16, 16)
                invc = invc_v[hh, sl]
                planes = [(idxr_v[ab, hh, sl], idxc_v[ab, hh, sl],
                           msk_v[ab, hh, sl]) for ab in range(4)]
                for ci in range(16):
                    csplat = jnp.full((16,), ci, jnp.int32)
                    acc = jnp.zeros((16,), jnp.float32)
                    for (rv, cv, mf) in planes:
                        g = plsc.load_gather(cond_v, [csplat, rv, cv])
                        acc = acc + g * mf
                    out_v[ci, hh, sl] = acc * invc
                return carry
            lax.fori_loop(0, 120, gath, 0)

            wrs = [pltpu.async_copy(
                out_v.at[ci], out_hbm.at[ch0 + ci, pl.ds(b0, 8), :], sem)
                for ci in range(16)]
            for cp in wrs:
                cp.wait()


@jax.jit
def kernel(content, mask, condition):
    c = content[0]
    m = mask[0]
    cd = condition[0]
    simi, code = pl.pallas_call(
        _sim_body,
        grid=(_NST,),
        in_specs=[
            pl.BlockSpec((_CCH, _H, _H), lambda i: (i, 0, 0)),
            pl.BlockSpec((_CCH, _H, _H), lambda i: (i, 0, 0)),
            pl.BlockSpec((_CCH, _H, _H), lambda i: (i, 0, 0)),
        ],
        out_specs=[
            pl.BlockSpec((_H, _H), lambda i: (0, 0)),
            pl.BlockSpec((_L, _L), lambda i: (0, 0)),
        ],
        out_shape=[
            jax.ShapeDtypeStruct((_H, _H), jnp.float32),
            jax.ShapeDtypeStruct((_L, _L), jnp.int32),
        ],
        scratch_shapes=[pltpu.VMEM((19, 226, 226), jnp.float32)],
    )(c, m, cd)

    mesh = plsc.VectorSubcoreMesh(core_axis_name="c", subcore_axis_name="s")
    sc_fold = functools.partial(
        pl.kernel, mesh=mesh,
        out_type=jax.ShapeDtypeStruct((32, 8 * _NBAND, 240), jnp.float32),
        scratch_types=[
            pltpu.VMEM((3 * _L,), jnp.int32),
            pltpu.VMEM((16, 10, _H), jnp.float32),
            pltpu.VMEM((16, 8, 240), jnp.float32),
            pltpu.VMEM((4, 8, 240), jnp.int32),
            pltpu.VMEM((4, 8, 240), jnp.int32),
            pltpu.VMEM((4, 8, 240), jnp.float32),
            pltpu.VMEM((8, 240), jnp.float32),
            pltpu.SemaphoreType.DMA,
        ],
        compiler_params=pltpu.CompilerParams(
            use_tc_tiling_on_sc=False, needs_layout_passes=False),
    )(_sc_fold_body)
    mapped_pad = sc_fold(cd, code.reshape(-1))
    mapped = mapped_pad[:, :_H, :_H]

    simi_full = jnp.broadcast_to(simi[None, None], (1, 32, _H, _H))
    return mapped[None], simi_full


# speedup vs baseline: 1.2061x; 1.2061x over previous
import functools
import jax
import jax.numpy as jnp
from jax import lax
from jax.experimental import pallas as pl
from jax.experimental.pallas import tpu as pltpu
from jax.experimental.pallas import tpu_sc as plsc

_H = 228
_L = 56
_CCH = 4
_NST = 32 // _CCH
_NBAND = 29


def _mm(a, b):
    return jax.lax.dot_general(
        a, b, (((1,), (0,)), ((), ())),
        precision=jax.lax.Precision.HIGHEST,
        preferred_element_type=jnp.float32)


def _selmat():
    r = jax.lax.broadcasted_iota(jnp.int32, (_L, 226), 1)
    lh = jax.lax.broadcasted_iota(jnp.int32, (_L, 226), 0)
    d = r - 4 * lh
    return ((d == 0) | (d == 2) | (d == 4)).astype(jnp.float32)


def _foldmat():
    h = jax.lax.broadcasted_iota(jnp.int32, (_H, _L), 0)
    lh = jax.lax.broadcasted_iota(jnp.int32, (_H, _L), 1)
    d = h - 4 * lh
    return ((d >= 0) & (d <= 4)).astype(jnp.float32)


def _boxsum(R, A):
    return _mm(_mm(A, R), A.T)


def _expand(M, P):
    return _mm(_mm(P, M), P.T)


def _sim_body(content_ref, mask_ref, cond_ref, simi_ref, code_ref, acc_ref):
    i = pl.program_id(0)

    @pl.when(i == 0)
    def _():
        acc_ref[...] = jnp.zeros_like(acc_ref)

    c0 = content_ref[...]
    m0 = mask_ref[...]
    cd = cond_ref[...]
    mp = (m0 > 0).astype(jnp.float32)
    cm = c0 * mp
    c2 = cd * cd

    acc_ref[18] += jnp.sum(c0 * c0, axis=0)[:226, :226]
    for n in range(9):
        sh, sw = n // 3, n % 3
        cs = cd[:, sh:sh + 226, sw:sw + 226]
        acc_ref[n] += jnp.sum(cm[:, :226, :226] * cs, axis=0)
        acc_ref[9 + n] += jnp.sum(
            mp[:, :226, :226] * c2[:, sh:sh + 226, sw:sw + 226], axis=0)

    @pl.when(i == _NST - 1)
    def _():
        A = _selmat()
        P = _foldmat()
        eps = jnp.float32(1e-8)
        V = _boxsum(acc_ref[18], A)
        sqV = jnp.sqrt(V) + eps
        nvv = V / (sqV * sqV)
        best_val = jnp.full((_L, _L), -jnp.inf, jnp.float32)
        best_idx = jnp.zeros((_L, _L), jnp.int32)
        for n in range(9):
            D = _boxsum(acc_ref[n], A)
            U = _boxsum(acc_ref[9 + n], A)
            sqU = jnp.sqrt(U) + eps
            nuu = U / (sqU * sqU)
            nvu = D / (sqV * sqU)
            eud = jnp.sqrt(jnp.maximum(nvv + nuu - 2.0 * nvu, 0.0))
            sim = (2.0 - eud) * 0.5
            upd = sim > best_val
            best_val = jnp.where(upd, sim, best_val)
            best_idx = jnp.where(upd, n, best_idx)

        maxval = best_val
        maxidx = jnp.where(maxval > 0, best_idx, 0)

        idx_f = maxidx.astype(jnp.float32)
        pieces = []
        for i2 in range(8):
            idx_s = idx_f[:, i2 * 7:(i2 + 1) * 7]
            sa = idx_f[:, i2 * 4:min((i2 + 1) * 7, _L)]
            ssum = jnp.sum(sa)
            scnt = jnp.sum(sa > 0.0).astype(jnp.float32)
            smean = jnp.round(ssum / (scnt + jnp.float32(1e-8)))
            pieces.append(jnp.where(idx_s > 0.0, idx_s, smean))
        fidx = jnp.concatenate(pieces, axis=1).astype(jnp.int32)

        code_ref[...] = (fidx // 3) * 4 + (fidx % 3)

        cnt = jnp.maximum(_expand(jnp.ones((_L, _L), jnp.float32), P), 1.0)
        simi_ref[...] = _expand(maxval, P) / cnt


def _bcast_body(simi_ref, out_ref):
    out_ref[...] = jnp.broadcast_to(simi_ref[...][None], (_CCH, _H, _H))


def _nab(hh):
    return 4 if hh % 4 == 0 else 2


def _sc_fold_body(cond_hbm, code_hbm, out_hbm,
                  code_v, cond_v, out_v, idxr_v, idxc_v, msk_v, invc_v, sem):
    wid = lax.axis_index("s") * 2 + lax.axis_index("c")

    @pl.when(wid < _NBAND)
    def _():
        b0 = wid * 8
        r0 = jnp.minimum(b0, 218)
        l0 = jnp.clip(2 * wid - 1, 0, 53)

        pltpu.async_copy(code_hbm.at[pl.ds(l0 * _L, 3 * _L)], code_v, sem).wait()

        classes = ((4, 2, lambda j: 4 * j),
                   (2, 6, lambda j: j + 1 + (j >= 3).astype(jnp.int32)))

        for nab, nrows, hh_of in classes:
            def abody(j, carry, nab=nab, hh_of=hh_of):
                hh = hh_of(jnp.int32(j))
                h = b0 + hh
                lh0 = h >> 2

                @plsc.parallel_loop(0, 15)
                def _(k):
                    sl = pl.ds(k * 16, 16)
                    wv = lax.iota(jnp.int32, 16) + k * 16
                    cnt = jnp.zeros((16,), jnp.float32)
                    for ab in range(nab):
                        a, b = ab >> 1, ab & 1
                        la = lh0 - a
                        vr = (la >= 0) & (la <= 55) & ((h & 3) + 4 * a <= 4)
                        lw = (wv >> 2) - b
                        vc = (lw >= 0) & (lw <= 55) & ((wv & 3) + 4 * b <= 4)
                        valid = vc & vr
                        pos = jnp.clip((la - l0) * _L + lw, 0, 3 * _L - 1)
                        code = plsc.load_gather(code_v, [pos])
                        rvec = jnp.clip((h - r0) + (code >> 2), 0, 9)
                        cvec = jnp.clip(wv + (code & 3), 0, 227)
                        mf = jnp.where(valid, jnp.float32(1.0),
                                       jnp.float32(0.0))
                        idxr_v[ab, hh, sl] = rvec
                        idxc_v[ab, hh, sl] = cvec
                        msk_v[ab, hh, sl] = mf
                        cnt = cnt + mf
                    if nab == 2:
                        invc_v[hh, sl] = 1.0 / jnp.maximum(cnt, 1.0)
                    else:
                        invc_v[hh, sl] = 1.0 / jnp.maximum(cnt, 1.0)
                return carry
            lax.fori_loop(0, nrows, abody, 0)

        def hbody(half, carry):
            ch0 = half * 16
            cps = [pltpu.async_copy(
                cond_hbm.at[ch0 + ci, pl.ds(r0, 10), :], cond_v.at[ci], sem)
                for ci in range(16)]
            for cp in cps:
                cp.wait()

            for nab, nrows, hh_of in classes:
                def _chunk(hh, k, tail, nab=nab):
                    sl = pl.ds(k * 16, 16)
                    invc = invc_v[hh, sl]
                    planes = [(idxr_v[ab, hh, sl], idxc_v[ab, hh, sl],
                               msk_v[ab, hh, sl]) for ab in range(nab)]
                    for ci in range(16):
                        csplat = jnp.full((16,), ci, jnp.int32)
                        acc = jnp.zeros((16,), jnp.float32)
                        for (rv, cv, mf) in planes:
                            g = plsc.load_gather(cond_v, [csplat, rv, cv])
                            acc = acc + g * mf
                        res = acc * invc
                        if tail:
                            lane = lax.iota(jnp.int32, 16)
                            plsc.store_scatter(
                                out_v,
                                [csplat, jnp.full((16,), hh, jnp.int32),
                                 jnp.clip(lane + k * 16, 0, _H - 1)],
                                res, mask=lane < _H - k * 16)
                        else:
                            out_v[ci, hh, sl] = res

                def rbody(j, carry2, hh_of=hh_of):
                    hh = hh_of(jnp.int32(j))

                    @plsc.parallel_loop(0, 14)
                    def _(k):
                        _chunk(hh, k, False)

                    _chunk(hh, 14, True)
                    return carry2
                lax.fori_loop(0, nrows, rbody, 0)

            @pl.when(wid < _NBAND - 1)
            def _():
                wrs = [pltpu.async_copy(
                    out_v.at[ci], out_hbm.at[ch0 + ci, pl.ds(b0, 8), :], sem)
                    for ci in range(16)]
                for cp in wrs:
                    cp.wait()

            @pl.when(wid == _NBAND - 1)
            def _():
                wrs = [pltpu.async_copy(
                    out_v.at[ci, 0:4],
                    out_hbm.at[ch0 + ci, pl.ds(b0, 4), :], sem)
                    for ci in range(16)]
                for cp in wrs:
                    cp.wait()
            return carry
        lax.fori_loop(0, 2, hbody, 0)


@jax.jit
def kernel(content, mask, condition):
    c = content[0]
    m = mask[0]
    cd = condition[0]
    simi, code = pl.pallas_call(
        _sim_body,
        grid=(_NST,),
        in_specs=[
            pl.BlockSpec((_CCH, _H, _H), lambda i: (i, 0, 0)),
            pl.BlockSpec((_CCH, _H, _H), lambda i: (i, 0, 0)),
            pl.BlockSpec((_CCH, _H, _H), lambda i: (i, 0, 0)),
        ],
        out_specs=[
            pl.BlockSpec((_H, _H), lambda i: (0, 0)),
            pl.BlockSpec((_L, _L), lambda i: (0, 0)),
        ],
        out_shape=[
            jax.ShapeDtypeStruct((_H, _H), jnp.float32),
            jax.ShapeDtypeStruct((_L, _L), jnp.int32),
        ],
        scratch_shapes=[pltpu.VMEM((19, 226, 226), jnp.float32)],
    )(c, m, cd)

    mesh = plsc.VectorSubcoreMesh(core_axis_name="c", subcore_axis_name="s")
    sc_fold = functools.partial(
        pl.kernel, mesh=mesh,
        out_type=jax.ShapeDtypeStruct((32, _H, _H), jnp.float32),
        scratch_types=[
            pltpu.VMEM((3 * _L,), jnp.int32),
            pltpu.VMEM((16, 10, _H), jnp.float32),
            pltpu.VMEM((16, 8, _H), jnp.float32),
            pltpu.VMEM((4, 8, 240), jnp.int32),
            pltpu.VMEM((4, 8, 240), jnp.int32),
            pltpu.VMEM((4, 8, 240), jnp.float32),
            pltpu.VMEM((8, 240), jnp.float32),
            pltpu.SemaphoreType.DMA,
        ],
        compiler_params=pltpu.CompilerParams(
            use_tc_tiling_on_sc=False, needs_layout_passes=False),
    )(_sc_fold_body)
    mapped = sc_fold(cd, code.reshape(-1))

    simi_full = pl.pallas_call(
        _bcast_body,
        grid=(_NST,),
        in_specs=[pl.BlockSpec((_H, _H), lambda i: (0, 0))],
        out_specs=pl.BlockSpec((_CCH, _H, _H), lambda i: (i, 0, 0)),
        out_shape=jax.ShapeDtypeStruct((32, _H, _H), jnp.float32),
    )(simi)
    return mapped[None], simi_full[None]


# speedup vs baseline: 1.3371x; 1.1086x over previous
import functools
import jax
import jax.numpy as jnp
from jax import lax
from jax.experimental import pallas as pl
from jax.experimental.pallas import tpu as pltpu
from jax.experimental.pallas import tpu_sc as plsc

_H = 228
_L = 56
_CCH = 4
_NST = 32 // _CCH
_NBAND = 28


def _mm(a, b):
    return jax.lax.dot_general(
        a, b, (((1,), (0,)), ((), ())),
        precision=jax.lax.Precision.HIGHEST,
        preferred_element_type=jnp.float32)


def _selmat():
    r = jax.lax.broadcasted_iota(jnp.int32, (_L, 226), 1)
    lh = jax.lax.broadcasted_iota(jnp.int32, (_L, 226), 0)
    d = r - 4 * lh
    return ((d == 0) | (d == 2) | (d == 4)).astype(jnp.float32)


def _foldmat():
    h = jax.lax.broadcasted_iota(jnp.int32, (_H, _L), 0)
    lh = jax.lax.broadcasted_iota(jnp.int32, (_H, _L), 1)
    d = h - 4 * lh
    return ((d >= 0) & (d <= 4)).astype(jnp.float32)


def _boxsum(R, A):
    return _mm(_mm(A, R), A.T)


def _expand(M, P):
    return _mm(_mm(P, M), P.T)


def _sim_body(content_ref, mask_ref, cond_ref,
              simi_ref, code_ref, wrow_ref, condp_ref, acc_ref):
    condp_ref[...] = jnp.pad(cond_ref[...], ((0, 0), (0, 4), (0, 0)))
    i = pl.program_id(0)

    @pl.when(i == 0)
    def _():
        acc_ref[...] = jnp.zeros_like(acc_ref)

    c0 = content_ref[...]
    m0 = mask_ref[...]
    cd = cond_ref[...]
    mp = (m0 > 0).astype(jnp.float32)
    cm = c0 * mp
    c2 = cd * cd

    acc_ref[18] += jnp.sum(c0 * c0, axis=0)[:226, :226]
    for n in range(9):
        sh, sw = n // 3, n % 3
        cs = cd[:, sh:sh + 226, sw:sw + 226]
        acc_ref[n] += jnp.sum(cm[:, :226, :226] * cs, axis=0)
        acc_ref[9 + n] += jnp.sum(
            mp[:, :226, :226] * c2[:, sh:sh + 226, sw:sw + 226], axis=0)

    @pl.when(i == _NST - 1)
    def _():
        A = _selmat()
        P = _foldmat()
        eps = jnp.float32(1e-8)
        V = _boxsum(acc_ref[18], A)
        sqV = jnp.sqrt(V) + eps
        nvv = V / (sqV * sqV)
        best_val = jnp.full((_L, _L), -jnp.inf, jnp.float32)
        best_idx = jnp.zeros((_L, _L), jnp.int32)
        for n in range(9):
            D = _boxsum(acc_ref[n], A)
            U = _boxsum(acc_ref[9 + n], A)
            sqU = jnp.sqrt(U) + eps
            nuu = U / (sqU * sqU)
            nvu = D / (sqV * sqU)
            eud = jnp.sqrt(jnp.maximum(nvv + nuu - 2.0 * nvu, 0.0))
            sim = (2.0 - eud) * 0.5
            upd = sim > best_val
            best_val = jnp.where(upd, sim, best_val)
            best_idx = jnp.where(upd, n, best_idx)

        maxval = best_val
        maxidx = jnp.where(maxval > 0, best_idx, 0)

        idx_f = maxidx.astype(jnp.float32)
        pieces = []
        for i2 in range(8):
            idx_s = idx_f[:, i2 * 7:(i2 + 1) * 7]
            sa = idx_f[:, i2 * 4:min((i2 + 1) * 7, _L)]
            ssum = jnp.sum(sa)
            scnt = jnp.sum(sa > 0.0).astype(jnp.float32)
            smean = jnp.round(ssum / (scnt + jnp.float32(1e-8)))
            pieces.append(jnp.where(idx_s > 0.0, idx_s, smean))
        fidx = jnp.concatenate(pieces, axis=1).astype(jnp.int32)

        code_ref[...] = (fidx // 3) * 4 + (fidx % 3)

        cnt = jnp.maximum(_expand(jnp.ones((_L, _L), jnp.float32), P), 1.0)
        simi_ref[...] = _expand(maxval, P) / cnt

        fr = fidx[_L - 1:_L, :]
        n9 = jax.lax.broadcasted_iota(jnp.int32, (9, _L), 0)
        M9 = (jnp.broadcast_to(fr, (9, _L)) == n9).astype(jnp.float32)
        cw = jnp.maximum(_mm(jnp.ones((1, _L), jnp.float32), P.T), 1.0)
        wrow_ref[...] = _mm(M9, P.T) / cw


def _bcast_body(simi_ref, wrow_ref, cond_ref, mapped0_ref,
                simi_out_ref, tail_ref):
    simi_out_ref[...] = jnp.broadcast_to(simi_ref[...][None], (_CCH, _H, _H))
    acc = jnp.zeros((_CCH, _H), jnp.float32)
    for n in range(9):
        sh, sw = n // 3, n % 3
        row = cond_ref[:, sh, :]
        cs = jnp.pad(row[:, sw:], ((0, 0), (0, sw)))
        acc = acc + wrow_ref[n][None, :] * cs
    tail_ref[...] = jnp.concatenate(
        [acc[:, None, :], jnp.zeros((_CCH, 7, _H), jnp.float32)], axis=1)


def _nab(hh):
    return 4 if hh % 4 == 0 else 2


def _sc_fold_body(cond_hbm, code_hbm, out_hbm,
                  code_v, cond_v, out_v, idxr_v, idxc_v, msk_v, invc_v, sem):
    wid = lax.axis_index("s") * 2 + lax.axis_index("c")

    @pl.when(wid < _NBAND)
    def _():
        b0 = wid * 8
        r0 = jnp.minimum(b0, 216)

        pltpu.async_copy(code_hbm, code_v, sem).wait()

        classes = ((4, 2, lambda j: 4 * j),
                   (2, 6, lambda j: j + 1 + (j >= 3).astype(jnp.int32)))

        for nab, nrows, hh_of in classes:
            def abody(j, carry, nab=nab, hh_of=hh_of):
                hh = hh_of(jnp.int32(j))
                h = b0 + hh
                lh0 = h >> 2

                @plsc.parallel_loop(0, 15)
                def _(k):
                    sl = pl.ds(k * 16, 16)
                    wv = lax.iota(jnp.int32, 16) + k * 16
                    cnt = jnp.zeros((16,), jnp.float32)
                    for ab in range(nab):
                        a, b = ab >> 1, ab & 1
                        la = lh0 - a
                        vr = (la >= 0) & (la <= 55) & ((h & 3) + 4 * a <= 4)
                        lw = (wv >> 2) - b
                        vc = (lw >= 0) & (lw <= 55) & ((wv & 3) + 4 * b <= 4)
                        valid = vc & vr
                        pos = jnp.clip(la * _L + lw, 0, _L * _L - 1)
                        code = plsc.load_gather(code_v, [pos])
                        rvec = jnp.clip((h - r0) + (code >> 2), 0, 15)
                        cvec = jnp.clip(wv + (code & 3), 0, 227)
                        mf = jnp.where(valid, jnp.float32(1.0),
                                       jnp.float32(0.0))
                        idxr_v[ab, hh, sl] = rvec
                        idxc_v[ab, hh, sl] = cvec
                        msk_v[ab, hh, sl] = mf
                        cnt = cnt + mf
                    if nab == 2:
                        invc_v[hh, sl] = 1.0 / jnp.maximum(cnt, 1.0)
                    else:
                        invc_v[hh, sl] = 1.0 / jnp.maximum(cnt, 1.0)
                return carry
            lax.fori_loop(0, nrows, abody, 0)

        def hbody(half, carry):
            ch0 = half * 16
            pltpu.async_copy(
                cond_hbm.at[pl.ds(ch0, 16), pl.ds(r0, 16), :], cond_v,
                sem).wait()

            for nab, nrows, hh_of in classes:
                def _chunk(hh, k, tail, nab=nab):
                    sl = pl.ds(k * 16, 16)
                    invc = invc_v[hh, sl]
                    planes = [(idxr_v[ab, hh, sl], idxc_v[ab, hh, sl],
                               msk_v[ab, hh, sl]) for ab in range(nab)]
                    for ci in range(16):
                        csplat = jnp.full((16,), ci, jnp.int32)
                        acc = jnp.zeros((16,), jnp.float32)
                        for (rv, cv, mf) in planes:
                            g = plsc.load_gather(cond_v, [csplat, rv, cv])
                            acc = acc + g * mf
                        res = acc * invc
                        if tail:
                            lane = lax.iota(jnp.int32, 16)
                            plsc.store_scatter(
                                out_v,
                                [csplat, jnp.full((16,), hh, jnp.int32),
                                 jnp.clip(lane + k * 16, 0, _H - 1)],
                                res, mask=lane < _H - k * 16)
                        else:
                            out_v[ci, hh, sl] = res

                def rbody(j, carry2, hh_of=hh_of):
                    hh = hh_of(jnp.int32(j))

                    @plsc.parallel_loop(0, 14)
                    def _(k):
                        _chunk(hh, k, False)

                    _chunk(hh, 14, True)
                    return carry2
                lax.fori_loop(0, nrows, rbody, 0)

            pltpu.async_copy(
                out_v, out_hbm.at[pl.ds(ch0, 16), pl.ds(b0, 8), :],
                sem).wait()
            return carry
        lax.fori_loop(0, 2, hbody, 0)


@jax.jit
def kernel(content, mask, condition):
    c = content[0]
    m = mask[0]
    cd = condition[0]
    simi, code, wrow, condp = pl.pallas_call(
        _sim_body,
        grid=(_NST,),
        in_specs=[
            pl.BlockSpec((_CCH, _H, _H), lambda i: (i, 0, 0)),
            pl.BlockSpec((_CCH, _H, _H), lambda i: (i, 0, 0)),
            pl.BlockSpec((_CCH, _H, _H), lambda i: (i, 0, 0)),
        ],
        out_specs=[
            pl.BlockSpec((_H, _H), lambda i: (0, 0)),
            pl.BlockSpec((_L, _L), lambda i: (0, 0)),
            pl.BlockSpec((9, _H), lambda i: (0, 0)),
            pl.BlockSpec((_CCH, 232, _H), lambda i: (i, 0, 0)),
        ],
        out_shape=[
            jax.ShapeDtypeStruct((_H, _H), jnp.float32),
            jax.ShapeDtypeStruct((_L, _L), jnp.int32),
            jax.ShapeDtypeStruct((9, _H), jnp.float32),
            jax.ShapeDtypeStruct((32, 232, _H), jnp.float32),
        ],
        scratch_shapes=[pltpu.VMEM((19, 226, 226), jnp.float32)],
    )(c, m, cd)

    mesh = plsc.VectorSubcoreMesh(core_axis_name="c", subcore_axis_name="s")
    sc_fold = functools.partial(
        pl.kernel, mesh=mesh,
        out_type=jax.ShapeDtypeStruct((32, _H, _H), jnp.float32),
        scratch_types=[
            pltpu.VMEM((_L * _L,), jnp.int32),
            pltpu.VMEM((16, 16, _H), jnp.float32),
            pltpu.VMEM((16, 8, _H), jnp.float32),
            pltpu.VMEM((4, 8, 240), jnp.int32),
            pltpu.VMEM((4, 8, 240), jnp.int32),
            pltpu.VMEM((4, 8, 240), jnp.float32),
            pltpu.VMEM((8, 240), jnp.float32),
            pltpu.SemaphoreType.DMA,
        ],
        compiler_params=pltpu.CompilerParams(needs_layout_passes=False),
    )(_sc_fold_body)
    mapped0 = sc_fold(condp, code.reshape(-1))

    simi_full, mapped = pl.pallas_call(
        _bcast_body,
        grid=(_NST,),
        in_specs=[
            pl.BlockSpec((_H, _H), lambda i: (0, 0)),
            pl.BlockSpec((9, _H), lambda i: (0, 0)),
            pl.BlockSpec((_CCH, 8, _H), lambda i: (i, 28, 0)),
            pl.BlockSpec(memory_space=pl.ANY),
        ],
        out_specs=[
            pl.BlockSpec((_CCH, _H, _H), lambda i: (i, 0, 0)),
            pl.BlockSpec((_CCH, 8, _H), lambda i: (i, 28, 0)),
        ],
        out_shape=[
            jax.ShapeDtypeStruct((32, _H, _H), jnp.float32),
            jax.ShapeDtypeStruct((32, _H, _H), jnp.float32),
        ],
        input_output_aliases={3: 1},
    )(simi, wrow, cd, mapped0)
    return mapped[None], simi_full[None]


# speedup vs baseline: 1.7646x; 1.3197x over previous
import functools
import jax
import jax.numpy as jnp
from jax import lax
from jax.experimental import pallas as pl
from jax.experimental.pallas import tpu as pltpu
from jax.experimental.pallas import tpu_sc as plsc

_H = 228
_L = 56
_CCH = 8
_NST = 32 // _CCH
_NBAND = 29


def _mm(a, b):
    return jax.lax.dot_general(
        a, b, (((1,), (0,)), ((), ())),
        precision=jax.lax.Precision.HIGHEST,
        preferred_element_type=jnp.float32)


def _selmat():
    r = jax.lax.broadcasted_iota(jnp.int32, (_L, 226), 1)
    lh = jax.lax.broadcasted_iota(jnp.int32, (_L, 226), 0)
    d = r - 4 * lh
    return ((d == 0) | (d == 2) | (d == 4)).astype(jnp.float32)


def _foldmat():
    h = jax.lax.broadcasted_iota(jnp.int32, (_H, _L), 0)
    lh = jax.lax.broadcasted_iota(jnp.int32, (_H, _L), 1)
    d = h - 4 * lh
    return ((d >= 0) & (d <= 4)).astype(jnp.float32)


def _boxsum(R, A):
    return _mm(_mm(A, R), A.T)


def _expand(M, P):
    return _mm(_mm(P, M), P.T)


def _sim_body(content_ref, mask_ref, cond_ref, simi_ref, code_ref, acc_ref):
    i = pl.program_id(0)

    @pl.when(i == 0)
    def _():
        acc_ref[...] = jnp.zeros_like(acc_ref)

    c0 = content_ref[...]
    m0 = mask_ref[...]
    cd = cond_ref[...]
    mp = (m0 > 0).astype(jnp.float32)
    cm = c0 * mp
    c2 = cd * cd

    acc_ref[18] += jnp.sum(c0 * c0, axis=1)[:226, :226]
    for n in range(9):
        sh, sw = n // 3, n % 3
        cs = cd[sh:sh + 226, :, sw:sw + 226]
        acc_ref[n] += jnp.sum(cm[:226, :, :226] * cs, axis=1)
        acc_ref[9 + n] += jnp.sum(
            mp[:226, :, :226] * c2[sh:sh + 226, :, sw:sw + 226], axis=1)

    @pl.when(i == _NST - 1)
    def _():
        A = _selmat()
        P = _foldmat()
        eps = jnp.float32(1e-8)
        V = _boxsum(acc_ref[18], A)
        sqV = jnp.sqrt(V) + eps
        nvv = V / (sqV * sqV)
        best_val = jnp.full((_L, _L), -jnp.inf, jnp.float32)
        best_idx = jnp.zeros((_L, _L), jnp.int32)
        for n in range(9):
            D = _boxsum(acc_ref[n], A)
            U = _boxsum(acc_ref[9 + n], A)
            sqU = jnp.sqrt(U) + eps
            nuu = U / (sqU * sqU)
            nvu = D / (sqV * sqU)
            eud = jnp.sqrt(jnp.maximum(nvv + nuu - 2.0 * nvu, 0.0))
            sim = (2.0 - eud) * 0.5
            upd = sim > best_val
            best_val = jnp.where(upd, sim, best_val)
            best_idx = jnp.where(upd, n, best_idx)

        maxval = best_val
        maxidx = jnp.where(maxval > 0, best_idx, 0)

        idx_f = maxidx.astype(jnp.float32)
        pieces = []
        for i2 in range(8):
            idx_s = idx_f[:, i2 * 7:(i2 + 1) * 7]
            sa = idx_f[:, i2 * 4:min((i2 + 1) * 7, _L)]
            ssum = jnp.sum(sa)
            scnt = jnp.sum(sa > 0.0).astype(jnp.float32)
            smean = jnp.round(ssum / (scnt + jnp.float32(1e-8)))
            pieces.append(jnp.where(idx_s > 0.0, idx_s, smean))
        fidx = jnp.concatenate(pieces, axis=1).astype(jnp.int32)

        code_ref[...] = (fidx // 3) * 4 + (fidx % 3)

        cnt = jnp.maximum(_expand(jnp.ones((_L, _L), jnp.float32), P), 1.0)
        simi_ref[...] = _expand(maxval, P) / cnt


def _bcast_body(simi_ref, out_ref):
    out_ref[...] = jnp.broadcast_to(
        simi_ref[...][:, None, :], (_H, _CCH, _H))


def _nab(hh):
    return 4 if hh % 4 == 0 else 2


def _sc_fold_body(cond_hbm, code_hbm, out_hbm,
                  code_v, cond_v, out_v, idxr_v, idxc_v, msk_v, invc_v, sem):
    wid = lax.axis_index("s") * 2 + lax.axis_index("c")

    @pl.when(wid < _NBAND)
    def _():
        b0 = wid * 8
        r0 = jnp.minimum(b0, 212)

        pltpu.async_copy(code_hbm, code_v, sem).wait()

        classes = ((4, 2, lambda j: 4 * j),
                   (2, 6, lambda j: j + 1 + (j >= 3).astype(jnp.int32)))

        for nab, nrows, hh_of in classes:
            def abody(j, carry, nab=nab, hh_of=hh_of):
                hh = hh_of(jnp.int32(j))
                h = b0 + hh
                lh0 = h >> 2

                @plsc.parallel_loop(0, 15)
                def _(k):
                    sl = pl.ds(k * 16, 16)
                    wv = lax.iota(jnp.int32, 16) + k * 16
                    cnt = jnp.zeros((16,), jnp.float32)
                    for ab in range(nab):
                        a, b = ab >> 1, ab & 1
                        la = lh0 - a
                        vr = (la >= 0) & (la <= 55) & ((h & 3) + 4 * a <= 4)
                        lw = (wv >> 2) - b
                        vc = (lw >= 0) & (lw <= 55) & ((wv & 3) + 4 * b <= 4)
                        valid = vc & vr
                        pos = jnp.clip(la * _L + lw, 0, _L * _L - 1)
                        code = plsc.load_gather(code_v, [pos])
                        rvec = jnp.clip((h - r0) + (code >> 2), 0, 15)
                        cvec = jnp.clip(wv + (code & 3), 0, 227)
                        mf = jnp.where(valid, jnp.float32(1.0),
                                       jnp.float32(0.0))
                        idxr_v[ab, hh, sl] = rvec
                        idxc_v[ab, hh, sl] = cvec
                        msk_v[ab, hh, sl] = mf
                        cnt = cnt + mf
                    invc_v[hh, sl] = 1.0 / jnp.maximum(cnt, 1.0)
                return carry
            lax.fori_loop(0, nrows, abody, 0)

        def hbody(half, carry):
            ch0 = half * 16
            pltpu.async_copy(
                cond_hbm.at[pl.ds(r0, 16), pl.ds(ch0, 16), :], cond_v,
                sem).wait()

            for nab, nrows, hh_of in classes:
                def _chunk(hh, k, tail, nab=nab):
                    sl = pl.ds(k * 16, 16)
                    invc = invc_v[hh, sl]
                    planes = [(idxr_v[ab, hh, sl], idxc_v[ab, hh, sl],
                               msk_v[ab, hh, sl]) for ab in range(nab)]
                    for ci in range(16):
                        csplat = jnp.full((16,), ci, jnp.int32)
                        acc = jnp.zeros((16,), jnp.float32)
                        for (rv, cv, mf) in planes:
                            g = plsc.load_gather(cond_v, [rv, csplat, cv])
                            acc = acc + g * mf
                        res = acc * invc
                        if tail:
                            lane = lax.iota(jnp.int32, 16)
                            plsc.store_scatter(
                                out_v,
                                [jnp.full((16,), hh, jnp.int32), csplat,
                                 jnp.clip(lane + k * 16, 0, _H - 1)],
                                res, mask=lane < _H - k * 16)
                        else:
                            out_v[hh, ci, sl] = res

                def rbody(j, carry2, hh_of=hh_of):
                    hh = hh_of(jnp.int32(j))

                    @plsc.parallel_loop(0, 14)
                    def _(k):
                        _chunk(hh, k, False)

                    _chunk(hh, 14, True)
                    return carry2
                lax.fori_loop(0, nrows, rbody, 0)

            @pl.when(wid < _NBAND - 1)
            def _():
                pltpu.async_copy(
                    out_v, out_hbm.at[pl.ds(b0, 8), pl.ds(ch0, 16), :],
                    sem).wait()

            @pl.when(wid == _NBAND - 1)
            def _():
                pltpu.async_copy(
                    out_v.at[0:4],
                    out_hbm.at[pl.ds(b0, 4), pl.ds(ch0, 16), :],
                    sem).wait()
            return carry
        lax.fori_loop(0, 2, hbody, 0)


@jax.jit
def kernel(content, mask, condition):
    c = jnp.transpose(content[0], (1, 0, 2))
    m = jnp.transpose(mask[0], (1, 0, 2))
    cd = jnp.transpose(condition[0], (1, 0, 2))
    simi, code = pl.pallas_call(
        _sim_body,
        grid=(_NST,),
        in_specs=[
            pl.BlockSpec((_H, _CCH, _H), lambda i: (0, i, 0)),
            pl.BlockSpec((_H, _CCH, _H), lambda i: (0, i, 0)),
            pl.BlockSpec((_H, _CCH, _H), lambda i: (0, i, 0)),
        ],
        out_specs=[
            pl.BlockSpec((_H, _H), lambda i: (0, 0)),
            pl.BlockSpec((_L, _L), lambda i: (0, 0)),
        ],
        out_shape=[
            jax.ShapeDtypeStruct((_H, _H), jnp.float32),
            jax.ShapeDtypeStruct((_L, _L), jnp.int32),
        ],
        scratch_shapes=[pltpu.VMEM((19, 226, 226), jnp.float32)],
    )(c, m, cd)

    mesh = plsc.VectorSubcoreMesh(core_axis_name="c", subcore_axis_name="s")
    sc_fold = functools.partial(
        pl.kernel, mesh=mesh,
        out_type=jax.ShapeDtypeStruct((_H, 32, _H), jnp.float32),
        scratch_types=[
            pltpu.VMEM((_L * _L,), jnp.int32),
            pltpu.VMEM((16, 16, _H), jnp.float32),
            pltpu.VMEM((8, 16, _H), jnp.float32),
            pltpu.VMEM((4, 8, 240), jnp.int32),
            pltpu.VMEM((4, 8, 240), jnp.int32),
            pltpu.VMEM((4, 8, 240), jnp.float32),
            pltpu.VMEM((8, 240), jnp.float32),
            pltpu.SemaphoreType.DMA,
        ],
        compiler_params=pltpu.CompilerParams(needs_layout_passes=False),
    )(_sc_fold_body)
    mapped_t = sc_fold(cd, code.reshape(-1))

    simi_t = pl.pallas_call(
        _bcast_body,
        grid=(_NST,),
        in_specs=[pl.BlockSpec((_H, _H), lambda i: (0, 0))],
        out_specs=pl.BlockSpec((_H, _CCH, _H), lambda i: (0, i, 0)),
        out_shape=jax.ShapeDtypeStruct((_H, 32, _H), jnp.float32),
    )(simi)

    mapped = jnp.transpose(mapped_t, (1, 0, 2))[None]
    simi_full = jnp.transpose(simi_t, (1, 0, 2))[None]
    return mapped, simi_full
